# baseline (speedup 1.0000x reference)
import functools

import jax
import jax.numpy as jnp
from jax import lax
from jax.experimental import pallas as pl
from jax.experimental.pallas import tpu as pltpu
from jax.experimental.pallas import tpu_sc as plsc

B, S, D = 4, 8192, 1024
NC, NS = 2, 16
NW = NC * NS
SEQ_PER_W = S // NW
R = 8
NCH = SEQ_PER_W // R
NPAIR = NCH // 2
VPR = D // 16

_mesh = plsc.VectorSubcoreMesh(core_axis_name="c", subcore_axis_name="s")


def _vmem():
    return pltpu.VMEM((R, D), jnp.float32)


@functools.partial(
    pl.kernel,
    mesh=_mesh,
    out_type=jax.ShapeDtypeStruct((B, S, D), jnp.float32),
    scratch_types=(
        [_vmem() for _ in range(2)]
        + [_vmem() for _ in range(2 * B)]
        + [pltpu.SemaphoreType.DMA] * 2
        + [pltpu.SemaphoreType.DMA] * (2 * B)
        + [pltpu.SemaphoreType.DMA] * (2 * B)
    ),
)
def _sc_add(x_hbm, pos_hbm, out_hbm, pos_a, pos_b,
            xa0, xa1, xa2, xa3, xb0, xb1, xb2, xb3,
            spa, spb,
            sia0, sia1, sia2, sia3, sib0, sib1, sib2, sib3,
            soa0, soa1, soa2, soa3, sob0, sob1, sob2, sob3):
    poss = (pos_a, pos_b)
    sps = (spa, spb)
    bufs = ((xa0, xa1, xa2, xa3), (xb0, xb1, xb2, xb3))
    sins = ((sia0, sia1, sia2, sia3), (sib0, sib1, sib2, sib3))
    souts = ((soa0, soa1, soa2, soa3), (sob0, sob1, sob2, sob3))
    wid = lax.axis_index("s") * NC + lax.axis_index("c")
    seq0 = wid * SEQ_PER_W

    def issue_ins(p, row):
        for u in range(B):
            pltpu.async_copy(x_hbm.at[u, pl.ds(row, R)], bufs[p][u],
                             sins[p][u])

    def wait_ins(p, row):
        for u in range(B):
            pltpu.make_async_copy(x_hbm.at[u, pl.ds(row, R)], bufs[p][u],
                                  sins[p][u]).wait()

    def issue_outs(p, row):
        for u in range(B):
            pltpu.async_copy(bufs[p][u], out_hbm.at[u, pl.ds(row, R)],
                             souts[p][u])

    def wait_outs(p, row):
        for u in range(B):
            pltpu.make_async_copy(bufs[p][u], out_hbm.at[u, pl.ds(row, R)],
                                  souts[p][u]).wait()

    pltpu.async_copy(pos_hbm.at[pl.ds(seq0, R)], pos_a, spa)
    pltpu.async_copy(pos_hbm.at[pl.ds(seq0 + R, R)], pos_b, spb)
    issue_ins(0, seq0)
    issue_ins(1, seq0 + R)

    def pair_body(i2, carry):
        for p in range(2):
            row0 = seq0 + (2 * i2 + p) * R
            wait_ins(p, row0)
            pltpu.make_async_copy(pos_hbm.at[pl.ds(row0, R)], poss[p],
                                  sps[p]).wait()

            def add_row(r, c2, _p=p):
                G = 8
                for g in range(VPR // G):
                    sls = [pl.ds((g * G + j) * 16, 16) for j in range(G)]
                    vals = [poss[_p][r, sl] for sl in sls]
                    for sl, v in zip(sls, vals):
                        for u in range(B):
                            plsc.addupdate(bufs[_p][u].at[r, sl], v)
                return c2

            lax.fori_loop(0, R, add_row, 0)
            issue_outs(p, row0)

            if p == 0:
                @pl.when(i2 > 0)
                def _(_row0=row0):
                    wait_outs(1, _row0 - R)
                    issue_ins(1, _row0 + R)
            else:
                wait_outs(0, row0 - R)

                @pl.when(i2 + 1 < NPAIR)
                def _(_row0=row0):
                    issue_ins(0, _row0 + R)
            @pl.when(i2 + 1 < NPAIR)
            def _(_row0=row0, _p=p):
                pltpu.async_copy(pos_hbm.at[pl.ds(_row0 + 2 * R, R)],
                                 poss[_p], sps[_p])
        return carry

    lax.fori_loop(0, NPAIR, pair_body, 0)
    wait_outs(1, seq0 + (NCH - 1) * R)


def kernel(x, pos_table):
    return _sc_add(x, pos_table)


# speedup vs baseline: 1.3874x; 1.3874x over previous
import functools

import jax
import jax.numpy as jnp
from jax import lax
from jax.experimental import pallas as pl
from jax.experimental.pallas import tpu as pltpu
from jax.experimental.pallas import tpu_sc as plsc

B, S, D = 4, 8192, 1024
NC, NS = 2, 16
NW = NC * NS
SEQ_PER_W = S // NW
R = 16
NCH = SEQ_PER_W // R
NPAIR = NCH // 2
VPR = D // 16

_mesh = plsc.VectorSubcoreMesh(core_axis_name="c", subcore_axis_name="s")


@functools.partial(
    pl.kernel,
    mesh=_mesh,
    out_type=jax.ShapeDtypeStruct((B, S, D), jnp.float32),
    scratch_types=[
        pltpu.VMEM((R, D), jnp.float32),
        pltpu.VMEM((R, D), jnp.float32),
        pltpu.VMEM((R, D), jnp.float32),
        pltpu.VMEM((R, D), jnp.float32),
        pltpu.VMEM((R, D), jnp.float32),
        pltpu.VMEM((R, D), jnp.float32),
        pltpu.SemaphoreType.DMA,
        pltpu.SemaphoreType.DMA,
        pltpu.SemaphoreType.DMA,
        pltpu.SemaphoreType.DMA,
        pltpu.SemaphoreType.DMA,
        pltpu.SemaphoreType.DMA,
        pltpu.SemaphoreType.DMA,
        pltpu.SemaphoreType.DMA,
        pltpu.SemaphoreType.DMA,
        pltpu.SemaphoreType.DMA,
    ],
)
def _sc_add(x_hbm, pos_hbm, out_hbm, pos_a, pos_b, xb0, xb1, xb2, xb3,
            spa, spb, si0, si1, si2, si3, so0, so1, so2, so3):
    bufs = (xb0, xb1, xb2, xb3)
    sins = (si0, si1, si2, si3)
    souts = (so0, so1, so2, so3)
    poss = (pos_a, pos_b)
    sps = (spa, spb)
    wid = lax.axis_index("s") * NC + lax.axis_index("c")
    seq0 = wid * SEQ_PER_W

    pltpu.async_copy(pos_hbm.at[pl.ds(seq0, R)], pos_a, spa)
    pltpu.async_copy(pos_hbm.at[pl.ds(seq0 + R, R)], pos_b, spb)
    for u in range(B):
        pltpu.async_copy(x_hbm.at[u, pl.ds(seq0, R)], bufs[u], sins[u])

    def pair_body(i2, carry):
        for half in range(2):
            row0 = seq0 + (2 * i2 + half) * R
            pos_v = poss[half]
            for u in range(B):
                pltpu.make_async_copy(
                    x_hbm.at[u, pl.ds(row0, R)], bufs[u], sins[u]).wait()
                if u == 0:
                    pltpu.make_async_copy(
                        pos_hbm.at[pl.ds(row0, R)], pos_v, sps[half]).wait()

                def add_row(r, c2, _buf=bufs[u], _pos=pos_v):
                    G = 16
                    NG = VPR // G

                    def slices(g):
                        return [pl.ds((g * G + j) * 16, 16) for j in range(G)]

                    cur_sls = slices(0)
                    cur_vals = [_pos[r, sl] for sl in cur_sls]
                    for g in range(NG):
                        if g + 1 < NG:
                            nxt_sls = slices(g + 1)
                            nxt_vals = [_pos[r, sl] for sl in nxt_sls]
                        for sl, v in zip(cur_sls, cur_vals):
                            plsc.addupdate(_buf.at[r, sl], v)
                        if g + 1 < NG:
                            cur_sls, cur_vals = nxt_sls, nxt_vals
                    return c2

                lax.fori_loop(0, R, add_row, 0)
                pltpu.async_copy(
                    bufs[u], out_hbm.at[u, pl.ds(row0, R)], souts[u])

                pu = (u - 1) % B
                if u == 0:
                    if half == 0:
                        @pl.when(i2 > 0)
                        def _():
                            pltpu.make_async_copy(
                                bufs[pu], out_hbm.at[pu, pl.ds(row0, R)],
                                souts[pu]).wait()
                            pltpu.async_copy(
                                x_hbm.at[pu, pl.ds(row0, R)],
                                bufs[pu], sins[pu])
                    else:
                        pltpu.make_async_copy(
                            bufs[pu], out_hbm.at[pu, pl.ds(row0, R)],
                            souts[pu]).wait()
                        pltpu.async_copy(
                            x_hbm.at[pu, pl.ds(row0, R)], bufs[pu], sins[pu])
                else:
                    pltpu.make_async_copy(
                        bufs[pu], out_hbm.at[pu, pl.ds(row0, R)],
                        souts[pu]).wait()
                    if half == 0:
                        pltpu.async_copy(
                            x_hbm.at[pu, pl.ds(row0 + R, R)],
                            bufs[pu], sins[pu])
                    else:
                        @pl.when(i2 + 1 < NPAIR)
                        def _():
                            pltpu.async_copy(
                                x_hbm.at[pu, pl.ds(row0 + R, R)],
                                bufs[pu], sins[pu])
            @pl.when(i2 + 1 < NPAIR)
            def _(_row0=row0, _half=half):
                pltpu.async_copy(
                    pos_hbm.at[pl.ds(_row0 + 2 * R, R)], poss[_half],
                    sps[_half])
        return carry

    lax.fori_loop(0, NPAIR, pair_body, 0)
    pltpu.make_async_copy(
        xb3, out_hbm.at[B - 1, pl.ds(seq0 + (NCH - 1) * R, R)], so3).wait()


def kernel(x, pos_table):
    return _sc_add(x, pos_table)
